# Initial kernel scaffold; baseline (speedup 1.0000x reference)
import jax
import jax.numpy as jnp
from jax.experimental import pallas as pl


def kernel(Uu, rest_positions, pair_v, pair_e0, pair_e1):
    raise NotImplementedError("write your pallas kernel here")


# speedup vs baseline: 3.3701x; 3.3701x over previous
import jax
import jax.numpy as jnp
from jax.experimental import pallas as pl
from jax.experimental.pallas import tpu as pltpu

DHAT2 = 0.05 * 0.05
EPS = 1e-12

N_CORES = 2
B_PAIRS = 2048
U = 4


def _coords_kernel(rest_ref, uu_ref, out_ref):
    out_ref[...] = rest_ref[...] + uu_ref[...]


def _energy_kernel(iv_ref, ie0_ref, ie1_ref, coords_ref, out_ref):
    b = pl.program_id(1)

    @pl.when(b == 0)
    def _init():
        out_ref[...] = jnp.zeros_like(out_ref)

    lane = jax.lax.broadcasted_iota(jnp.int32, (8, 128), 1)
    parity_even = (lane & 1) == 0
    lane0 = lane == 0

    def pair_sum(x):
        return x + jnp.where(parity_even, pltpu.roll(x, 127, axis=1),
                             pltpu.roll(x, 1, axis=1))

    def gather_point(ref, k):
        i = ref[0, 0, k]
        r = jax.lax.shift_right_logical(i, 6)
        c = (i & 63) * 2
        slab = coords_ref[r]
        return pltpu.roll(slab, (128 - c) & 127, axis=1)

    def chunk(base, acc):
        vs, as_, bs = [], [], []
        for j in range(8):
            k = base + j
            vs.append(gather_point(iv_ref, k))
            as_.append(gather_point(ie0_ref, k))
            bs.append(gather_point(ie1_ref, k))
        cv = jnp.concatenate(vs, axis=0)
        ca = jnp.concatenate(as_, axis=0)
        cb = jnp.concatenate(bs, axis=0)

        ab = cb - ca
        ap = cv - ca
        denom = pair_sum(ab * ab)
        dotv = pair_sum(ap * ab)
        t = jnp.clip(dotv / jnp.maximum(denom, EPS), 0.0, 1.0)
        diff = ap - t * ab
        d2 = pair_sum(diff * diff)
        d2s = jnp.maximum(d2, EPS)
        barrier = -jnp.square(d2s - DHAT2) * jnp.log(d2s / DHAT2)
        return acc + jnp.where((d2 < DHAT2) & lane0, barrier, 0.0)

    def body(it, acc):
        base = it * (8 * U)
        for u in range(U):
            acc = chunk(base + 8 * u, acc)
        return acc

    acc = jax.lax.fori_loop(0, B_PAIRS // (8 * U), body,
                            jnp.zeros((8, 128), jnp.float32))
    out_ref[...] += acc


def kernel(Uu, rest_positions, pair_v, pair_e0, pair_e1):
    n_verts = rest_positions.shape[0]
    n_pairs = pair_v.shape[0]
    flat_rows = n_verts * 2 // 128
    blocks = n_pairs // B_PAIRS
    inner = blocks // N_CORES

    rest_flat = rest_positions.reshape(flat_rows, 128)
    uu_flat = Uu.reshape(flat_rows, 128)
    coords = pl.pallas_call(
        _coords_kernel,
        out_shape=jax.ShapeDtypeStruct((flat_rows, 128), jnp.float32),
    )(rest_flat, uu_flat)
    coords3d = coords.reshape(flat_rows, 1, 128)

    pv = pair_v.astype(jnp.int32).reshape(blocks, 1, B_PAIRS)
    pe0 = pair_e0.astype(jnp.int32).reshape(blocks, 1, B_PAIRS)
    pe1 = pair_e1.astype(jnp.int32).reshape(blocks, 1, B_PAIRS)

    idx_spec = pl.BlockSpec((1, 1, B_PAIRS),
                            lambda c, i: (c * inner + i, 0, 0),
                            memory_space=pltpu.SMEM)
    partial = pl.pallas_call(
        _energy_kernel,
        grid=(N_CORES, inner),
        in_specs=[idx_spec, idx_spec, idx_spec,
                  pl.BlockSpec((flat_rows, 1, 128), lambda c, i: (0, 0, 0))],
        out_specs=pl.BlockSpec((1, 8, 128), lambda c, i: (c, 0, 0)),
        out_shape=jax.ShapeDtypeStruct((N_CORES, 8, 128), jnp.float32),
        compiler_params=pltpu.CompilerParams(
            dimension_semantics=("parallel", "arbitrary")),
    )(pv, pe0, pe1, coords3d)
    return jnp.sum(partial)


# speedup vs baseline: 4.8743x; 1.4463x over previous
import jax
import jax.numpy as jnp
from jax.experimental import pallas as pl
from jax.experimental.pallas import tpu as pltpu

DHAT2 = 0.05 * 0.05
EPS = 1e-12

N_CORES = 2
B_PAIRS = 2048
U = 8


def _coords_kernel(rest_ref, uu_ref, out_ref):
    out_ref[...] = rest_ref[...] + uu_ref[...]


def _energy_kernel(rv_ref, sv_ref, r0_ref, s0_ref, r1_ref, s1_ref,
                   coords_ref, out_ref):
    b = pl.program_id(1)

    @pl.when(b == 0)
    def _init():
        out_ref[...] = jnp.zeros_like(out_ref)

    lane = jax.lax.broadcasted_iota(jnp.int32, (8, 128), 1)
    parity_even = (lane & 1) == 0
    lane0 = lane == 0

    def pair_sum(x):
        return x + jnp.where(parity_even, pltpu.roll(x, 127, axis=1),
                             pltpu.roll(x, 1, axis=1))

    def gather_point(row_ref, sh_ref, k):
        slab = coords_ref[row_ref[0, 0, k]]
        return pltpu.roll(slab, sh_ref[0, 0, k], axis=1)

    def chunk(base, acc):
        vs, as_, bs = [], [], []
        for j in range(8):
            k = base + j
            vs.append(gather_point(rv_ref, sv_ref, k))
            as_.append(gather_point(r0_ref, s0_ref, k))
            bs.append(gather_point(r1_ref, s1_ref, k))
        cv = jnp.concatenate(vs, axis=0)
        ca = jnp.concatenate(as_, axis=0)
        cb = jnp.concatenate(bs, axis=0)

        ab = cb - ca
        ap = cv - ca
        denom = pair_sum(ab * ab)
        dotv = pair_sum(ap * ab)
        t = jnp.clip(dotv / jnp.maximum(denom, EPS), 0.0, 1.0)
        diff = ap - t * ab
        d2 = pair_sum(diff * diff)
        d2s = jnp.maximum(d2, EPS)
        barrier = -jnp.square(d2s - DHAT2) * jnp.log(d2s / DHAT2)
        return acc + jnp.where((d2 < DHAT2) & lane0, barrier, 0.0)

    def body(it, acc):
        base = it * (8 * U)
        for u in range(U):
            acc = chunk(base + 8 * u, acc)
        return acc

    acc = jax.lax.fori_loop(0, B_PAIRS // (8 * U), body,
                            jnp.zeros((8, 128), jnp.float32))
    out_ref[...] += acc


def kernel(Uu, rest_positions, pair_v, pair_e0, pair_e1):
    n_verts = rest_positions.shape[0]
    n_pairs = pair_v.shape[0]
    flat_rows = n_verts * 2 // 128
    blocks = n_pairs // B_PAIRS
    inner = blocks // N_CORES

    rest_flat = rest_positions.reshape(flat_rows, 128)
    uu_flat = Uu.reshape(flat_rows, 128)
    coords = pl.pallas_call(
        _coords_kernel,
        out_shape=jax.ShapeDtypeStruct((flat_rows, 128), jnp.float32),
    )(rest_flat, uu_flat)
    coords3d = coords.reshape(flat_rows, 1, 128)

    def row_shift(p):
        i = p.astype(jnp.int32)
        row = jax.lax.shift_right_logical(i, 6).reshape(blocks, 1, B_PAIRS)
        sh = ((128 - (i & 63) * 2) & 127).reshape(blocks, 1, B_PAIRS)
        return row, sh

    rv, sv = row_shift(pair_v)
    r0, s0 = row_shift(pair_e0)
    r1, s1 = row_shift(pair_e1)

    idx_spec = pl.BlockSpec((1, 1, B_PAIRS),
                            lambda c, i: (c * inner + i, 0, 0),
                            memory_space=pltpu.SMEM)
    partial = pl.pallas_call(
        _energy_kernel,
        grid=(N_CORES, inner),
        in_specs=[idx_spec] * 6 +
                 [pl.BlockSpec((flat_rows, 1, 128), lambda c, i: (0, 0, 0))],
        out_specs=pl.BlockSpec((1, 8, 128), lambda c, i: (c, 0, 0)),
        out_shape=jax.ShapeDtypeStruct((N_CORES, 8, 128), jnp.float32),
        compiler_params=pltpu.CompilerParams(
            dimension_semantics=("parallel", "arbitrary")),
    )(rv, sv, r0, s0, r1, s1, coords3d)
    return jnp.sum(partial)


# speedup vs baseline: 9.9829x; 2.0481x over previous
import jax
import jax.numpy as jnp
from jax.experimental import pallas as pl
from jax.experimental.pallas import tpu as pltpu

DHAT2 = 0.05 * 0.05
EPS = 1e-12

N_CORES = 2
B_PAIRS = 2048
U = 8
GROUPS = B_PAIRS // (8 * U)


def _coords_kernel(rest_ref, uu_ref, out_ref):
    out_ref[...] = rest_ref[...] + uu_ref[...]


def _energy_kernel(rv_ref, r0_ref, r1_ref, cv_ref, c0_ref, c1_ref,
                   coords_ref, out_ref, slv_ref, sl0_ref, sl1_ref):
    b = pl.program_id(1)

    @pl.when(b == 0)
    def _init():
        out_ref[...] = jnp.zeros_like(out_ref)

    lane = jax.lax.broadcasted_iota(jnp.int32, (8, 128), 1)
    parity_even = (lane & 1) == 0
    lane0 = lane == 0
    iota1 = lane & 1

    def pair_sum(x):
        return x + jnp.where(parity_even, pltpu.roll(x, 127, axis=1),
                             pltpu.roll(x, 1, axis=1))

    def compute(cv, ca, cb, acc):
        ab = cb - ca
        ap = cv - ca
        denom = pair_sum(ab * ab)
        dotv = pair_sum(ap * ab)
        t = jnp.clip(dotv / jnp.maximum(denom, EPS), 0.0, 1.0)
        diff = ap - t * ab
        d2 = pair_sum(diff * diff)
        d2s = jnp.maximum(d2, EPS)
        barrier = -jnp.square(d2s - DHAT2) * jnp.log(d2s / DHAT2)
        return acc + jnp.where((d2 < DHAT2) & lane0, barrier, 0.0)

    def body(it, acc):
        for h in range(2):
            base = (2 * it + h) * (8 * U)
            off = h * U
            for u in range(U):
                for j in range(8):
                    k = base + 8 * j + u
                    s = off + u
                    slv_ref[s, j:j + 1, :] = coords_ref[rv_ref[0, 0, k]]
                    sl0_ref[s, j:j + 1, :] = coords_ref[r0_ref[0, 0, k]]
                    sl1_ref[s, j:j + 1, :] = coords_ref[r1_ref[0, 0, k]]
        for h in range(2):
            ctv = cv_ref[2 * it + h]
            ct0 = c0_ref[2 * it + h]
            ct1 = c1_ref[2 * it + h]
            off = h * U
            for u in range(U):
                s = off + u
                liv = jnp.broadcast_to(ctv[:, u:u + 1], (8, 128)) + iota1
                li0 = jnp.broadcast_to(ct0[:, u:u + 1], (8, 128)) + iota1
                li1 = jnp.broadcast_to(ct1[:, u:u + 1], (8, 128)) + iota1
                cv = jnp.take_along_axis(slv_ref[s], liv, axis=1)
                ca = jnp.take_along_axis(sl0_ref[s], li0, axis=1)
                cb = jnp.take_along_axis(sl1_ref[s], li1, axis=1)
                acc = compute(cv, ca, cb, acc)
        return acc

    acc = jax.lax.fori_loop(0, GROUPS // 2, body,
                            jnp.zeros((8, 128), jnp.float32))
    out_ref[...] += acc


def kernel(Uu, rest_positions, pair_v, pair_e0, pair_e1):
    n_verts = rest_positions.shape[0]
    n_pairs = pair_v.shape[0]
    flat_rows = n_verts * 2 // 128
    blocks = n_pairs // B_PAIRS
    inner = blocks // N_CORES

    rest_flat = rest_positions.reshape(flat_rows, 128)
    uu_flat = Uu.reshape(flat_rows, 128)
    coords = pl.pallas_call(
        _coords_kernel,
        out_shape=jax.ShapeDtypeStruct((flat_rows, 128), jnp.float32),
    )(rest_flat, uu_flat)
    coords3d = coords.reshape(flat_rows, 1, 128)

    def row_lane(p):
        i = p.astype(jnp.int32)
        row = jax.lax.shift_right_logical(i, 6).reshape(blocks, 1, B_PAIRS)
        lanes = ((i & 63) * 2).reshape(blocks * GROUPS, 8, 8)
        return row, lanes

    rv, cv = row_lane(pair_v)
    r0, c0 = row_lane(pair_e0)
    r1, c1 = row_lane(pair_e1)

    row_spec = pl.BlockSpec((1, 1, B_PAIRS),
                            lambda c, i: (c * inner + i, 0, 0),
                            memory_space=pltpu.SMEM)
    lane_spec = pl.BlockSpec((GROUPS, 8, 8),
                             lambda c, i: (c * inner + i, 0, 0))
    partial = pl.pallas_call(
        _energy_kernel,
        grid=(N_CORES, inner),
        in_specs=[row_spec] * 3 + [lane_spec] * 3 +
                 [pl.BlockSpec((flat_rows, 1, 128), lambda c, i: (0, 0, 0))],
        out_specs=pl.BlockSpec((1, 8, 128), lambda c, i: (c, 0, 0)),
        out_shape=jax.ShapeDtypeStruct((N_CORES, 8, 128), jnp.float32),
        scratch_shapes=[pltpu.VMEM((2 * U, 8, 128), jnp.float32)] * 3,
        compiler_params=pltpu.CompilerParams(
            dimension_semantics=("parallel", "arbitrary")),
    )(rv, r0, r1, cv, c0, c1, coords3d)
    return jnp.sum(partial)


# speedup vs baseline: 10.0391x; 1.0056x over previous
import jax
import jax.numpy as jnp
from jax.experimental import pallas as pl
from jax.experimental.pallas import tpu as pltpu

DHAT2 = 0.05 * 0.05
EPS = 1e-12

N_CORES = 2
B_PAIRS = 8192
U = 8
NG = 2
GROUPS = B_PAIRS // (8 * U)


def _coords_kernel(rest_ref, uu_ref, out_ref):
    out_ref[...] = rest_ref[...] + uu_ref[...]


def _energy_kernel(rv_ref, r0_ref, r1_ref, cv_ref, c0_ref, c1_ref,
                   coords_ref, out_ref, slv_ref, sl0_ref, sl1_ref):
    b = pl.program_id(1)

    @pl.when(b == 0)
    def _init():
        out_ref[...] = jnp.zeros_like(out_ref)

    lane = jax.lax.broadcasted_iota(jnp.int32, (8, 128), 1)
    parity_even = (lane & 1) == 0
    lane0 = lane == 0
    iota1 = lane & 1

    def pair_sum(x):
        return x + jnp.where(parity_even, pltpu.roll(x, 127, axis=1),
                             pltpu.roll(x, 1, axis=1))

    def compute(cv, ca, cb, acc):
        ab = cb - ca
        ap = cv - ca
        denom = pair_sum(ab * ab)
        dotv = pair_sum(ap * ab)
        t = jnp.clip(dotv / jnp.maximum(denom, EPS), 0.0, 1.0)
        diff = ap - t * ab
        d2 = pair_sum(diff * diff)
        d2s = jnp.maximum(d2, EPS)
        barrier = -jnp.square(d2s - DHAT2) * jnp.log(d2s / DHAT2)
        return acc + jnp.where((d2 < DHAT2) & lane0, barrier, 0.0)

    def body(it, acc):
        for h in range(NG):
            base = (NG * it + h) * (8 * U)
            off = h * U
            for u in range(U):
                for j in range(8):
                    k = base + 8 * j + u
                    s = off + u
                    slv_ref[s, j:j + 1, :] = coords_ref[rv_ref[0, 0, k]]
                    sl0_ref[s, j:j + 1, :] = coords_ref[r0_ref[0, 0, k]]
                    sl1_ref[s, j:j + 1, :] = coords_ref[r1_ref[0, 0, k]]
        for h in range(NG):
            ctv = cv_ref[NG * it + h]
            ct0 = c0_ref[NG * it + h]
            ct1 = c1_ref[NG * it + h]
            off = h * U
            for u in range(U):
                s = off + u
                liv = jnp.broadcast_to(ctv[:, u:u + 1], (8, 128)) + iota1
                li0 = jnp.broadcast_to(ct0[:, u:u + 1], (8, 128)) + iota1
                li1 = jnp.broadcast_to(ct1[:, u:u + 1], (8, 128)) + iota1
                cv = jnp.take_along_axis(slv_ref[s], liv, axis=1)
                ca = jnp.take_along_axis(sl0_ref[s], li0, axis=1)
                cb = jnp.take_along_axis(sl1_ref[s], li1, axis=1)
                acc = compute(cv, ca, cb, acc)
        return acc

    acc = jax.lax.fori_loop(0, GROUPS // NG, body,
                            jnp.zeros((8, 128), jnp.float32))
    out_ref[...] += acc


def kernel(Uu, rest_positions, pair_v, pair_e0, pair_e1):
    n_verts = rest_positions.shape[0]
    n_pairs = pair_v.shape[0]
    flat_rows = n_verts * 2 // 128
    blocks = n_pairs // B_PAIRS
    inner = blocks // N_CORES

    rest_flat = rest_positions.reshape(flat_rows, 128)
    uu_flat = Uu.reshape(flat_rows, 128)
    coords = pl.pallas_call(
        _coords_kernel,
        out_shape=jax.ShapeDtypeStruct((flat_rows, 128), jnp.float32),
    )(rest_flat, uu_flat)
    coords3d = coords.reshape(flat_rows, 1, 128)

    def row_lane(p):
        i = p.astype(jnp.int32)
        row = jax.lax.shift_right_logical(i, 6).reshape(blocks, 1, B_PAIRS)
        lanes = ((i & 63) * 2).reshape(blocks * GROUPS, 8, 8)
        return row, lanes

    rv, cv = row_lane(pair_v)
    r0, c0 = row_lane(pair_e0)
    r1, c1 = row_lane(pair_e1)

    row_spec = pl.BlockSpec((1, 1, B_PAIRS),
                            lambda c, i: (c * inner + i, 0, 0),
                            memory_space=pltpu.SMEM)
    lane_spec = pl.BlockSpec((GROUPS, 8, 8),
                             lambda c, i: (c * inner + i, 0, 0))
    partial = pl.pallas_call(
        _energy_kernel,
        grid=(N_CORES, inner),
        in_specs=[row_spec] * 3 + [lane_spec] * 3 +
                 [pl.BlockSpec((flat_rows, 1, 128), lambda c, i: (0, 0, 0))],
        out_specs=pl.BlockSpec((1, 8, 128), lambda c, i: (c, 0, 0)),
        out_shape=jax.ShapeDtypeStruct((N_CORES, 8, 128), jnp.float32),
        scratch_shapes=[pltpu.VMEM((NG * U, 8, 128), jnp.float32)] * 3,
        compiler_params=pltpu.CompilerParams(
            dimension_semantics=("parallel", "arbitrary")),
    )(rv, r0, r1, cv, c0, c1, coords3d)
    return jnp.sum(partial)


# speedup vs baseline: 10.0409x; 1.0002x over previous
import jax
import jax.numpy as jnp
from jax.experimental import pallas as pl
from jax.experimental.pallas import tpu as pltpu

DHAT2 = 0.05 * 0.05
EPS = 1e-12

N_CORES = 2
B_PAIRS = 8192
U = 8
NG = 2
GROUPS = B_PAIRS // (8 * U)


def _coords_kernel(rest_ref, uu_ref, out_ref):
    out_ref[...] = rest_ref[...] + uu_ref[...]


def _energy_kernel(rv_ref, r0_ref, r1_ref, cv_ref, c0_ref, c1_ref,
                   coords_ref, out_ref, *slots):
    b = pl.program_id(1)

    @pl.when(b == 0)
    def _init():
        out_ref[...] = jnp.zeros_like(out_ref)

    lane = jax.lax.broadcasted_iota(jnp.int32, (8, 128), 1)
    parity_even = (lane & 1) == 0
    lane0 = lane == 0
    iota1 = lane & 1

    def pair_sum(x):
        return x + jnp.where(parity_even, pltpu.roll(x, 127, axis=1),
                             pltpu.roll(x, 1, axis=1))

    def compute(cv, ca, cb, acc):
        ab = cb - ca
        ap = cv - ca
        denom = pair_sum(ab * ab)
        dotv = pair_sum(ap * ab)
        t = jnp.clip(dotv / jnp.maximum(denom, EPS), 0.0, 1.0)
        diff = ap - t * ab
        d2 = pair_sum(diff * diff)
        d2s = jnp.maximum(d2, EPS)
        barrier = -jnp.square(d2s - DHAT2) * jnp.log(d2s / DHAT2)
        return acc + jnp.where((d2 < DHAT2) & lane0, barrier, 0.0)

    def p1_chunk(it, h, u):
        base = (NG * it + h) * (8 * U)
        slv_ref, sl0_ref, sl1_ref = slots[3 * h:3 * h + 3]
        for j in range(8):
            k = base + 8 * j + u
            slv_ref[u, j:j + 1, :] = coords_ref[rv_ref[0, 0, k]]
            sl0_ref[u, j:j + 1, :] = coords_ref[r0_ref[0, 0, k]]
            sl1_ref[u, j:j + 1, :] = coords_ref[r1_ref[0, 0, k]]

    def p2_chunk(h, u, cts, acc):
        ctv, ct0, ct1 = cts
        slv_ref, sl0_ref, sl1_ref = slots[3 * h:3 * h + 3]
        liv = jnp.broadcast_to(ctv[:, u:u + 1], (8, 128)) + iota1
        li0 = jnp.broadcast_to(ct0[:, u:u + 1], (8, 128)) + iota1
        li1 = jnp.broadcast_to(ct1[:, u:u + 1], (8, 128)) + iota1
        cv = jnp.take_along_axis(slv_ref[u], liv, axis=1)
        ca = jnp.take_along_axis(sl0_ref[u], li0, axis=1)
        cb = jnp.take_along_axis(sl1_ref[u], li1, axis=1)
        return compute(cv, ca, cb, acc)

    def body(it, acc):
        for h in range(NG):
            for u in range(U):
                p1_chunk(it, h, u)
        for h in range(NG):
            cts = (cv_ref[NG * it + h], c0_ref[NG * it + h],
                   c1_ref[NG * it + h])
            for u in range(U):
                acc = p2_chunk(h, u, cts, acc)
        return acc

    acc = jax.lax.fori_loop(0, GROUPS // NG, body,
                            jnp.zeros((8, 128), jnp.float32))
    out_ref[...] += acc


def kernel(Uu, rest_positions, pair_v, pair_e0, pair_e1):
    n_verts = rest_positions.shape[0]
    n_pairs = pair_v.shape[0]
    flat_rows = n_verts * 2 // 128
    blocks = n_pairs // B_PAIRS
    inner = blocks // N_CORES

    rest_flat = rest_positions.reshape(flat_rows, 128)
    uu_flat = Uu.reshape(flat_rows, 128)
    coords = pl.pallas_call(
        _coords_kernel,
        out_shape=jax.ShapeDtypeStruct((flat_rows, 128), jnp.float32),
    )(rest_flat, uu_flat)
    coords3d = coords.reshape(flat_rows, 1, 128)

    def row_lane(p):
        i = p.astype(jnp.int32)
        row = jax.lax.shift_right_logical(i, 6).reshape(blocks, 1, B_PAIRS)
        lanes = ((i & 63) * 2).reshape(blocks * GROUPS, 8, 8)
        return row, lanes

    rv, cv = row_lane(pair_v)
    r0, c0 = row_lane(pair_e0)
    r1, c1 = row_lane(pair_e1)

    row_spec = pl.BlockSpec((1, 1, B_PAIRS),
                            lambda c, i: (c * inner + i, 0, 0),
                            memory_space=pltpu.SMEM)
    lane_spec = pl.BlockSpec((GROUPS, 8, 8),
                             lambda c, i: (c * inner + i, 0, 0))
    partial = pl.pallas_call(
        _energy_kernel,
        grid=(N_CORES, inner),
        in_specs=[row_spec] * 3 + [lane_spec] * 3 +
                 [pl.BlockSpec((flat_rows, 1, 128), lambda c, i: (0, 0, 0))],
        out_specs=pl.BlockSpec((1, 8, 128), lambda c, i: (c, 0, 0)),
        out_shape=jax.ShapeDtypeStruct((N_CORES, 8, 128), jnp.float32),
        scratch_shapes=[pltpu.VMEM((U, 8, 128), jnp.float32)] * (3 * NG),
        compiler_params=pltpu.CompilerParams(
            dimension_semantics=("parallel", "arbitrary")),
    )(rv, r0, r1, cv, c0, c1, coords3d)
    return jnp.sum(partial)
